# baseline (speedup 1.0000x reference)
import functools

import jax
import jax.numpy as jnp
from jax import lax
from jax.experimental import pallas as pl
from jax.experimental.pallas import tpu as pltpu
from jax.experimental.pallas import tpu_sc as plsc

_NC, _NS = 2, 16
_NW = _NC * _NS


def _tc_fold_pair(lt1, lt2, pre1, pre2, W1, W2):
    D, V = lt1.shape
    P = pre1.shape[1]
    VB = 2048
    grid = ((V + VB - 1) // VB,)
    f32 = jnp.float32

    def body(lt1_r, lt2_r, p1_r, p2_r, w1_r, w2_r, out_r):
        def fused(lt_r, p_r, w_r):
            lt = jnp.transpose(lt_r[...], (1, 0))
            return lt + jnp.dot(p_r[...], w_r[...], preferred_element_type=f32)
        out_r[...] = jnp.concatenate(
            [fused(lt1_r, p1_r, w1_r), fused(lt2_r, p2_r, w2_r)], axis=1)

    blk_lt = pl.BlockSpec((D, VB), lambda i: (0, i))
    blk_p = pl.BlockSpec((VB, P), lambda i: (i, 0))
    blk_w = pl.BlockSpec((P, D), lambda i: (0, 0))
    blk_o = pl.BlockSpec((VB, 2 * D), lambda i: (i, 0))

    return pl.pallas_call(
        body,
        grid=grid,
        in_specs=[blk_lt, blk_lt, blk_p, blk_p, blk_w, blk_w],
        out_specs=blk_o,
        out_shape=jax.ShapeDtypeStruct((V, 2 * D), f32),
    )(lt1, lt2, pre1, pre2, W1, W2)


def _mesh():
    return plsc.VectorSubcoreMesh(core_axis_name="c", subcore_axis_name="s",
                                  num_cores=_NC, num_subcores=_NS)


def _sc_gather_x(s_tok, i_tok, X):
    B = s_tok.shape[0]
    W2 = X.shape[1]
    D = W2 // 2
    f32 = jnp.float32
    SUB = 128
    chunk = B // _NW
    nstep = chunk // SUB

    @functools.partial(
        pl.kernel,
        out_type=jax.ShapeDtypeStruct((B, D), f32),
        mesh=_mesh(),
        compiler_params=pltpu.CompilerParams(use_tc_tiling_on_sc=True),
        scratch_types=[
            pltpu.VMEM((SUB,), jnp.int32),
            pltpu.VMEM((SUB,), jnp.int32),
            pltpu.VMEM((SUB, W2), f32),
            pltpu.VMEM((SUB, W2), f32),
            pltpu.VMEM((SUB, D), f32),
            pltpu.SemaphoreType.DMA,
            pltpu.SemaphoreType.DMA,
        ],
    )
    def sc_k(s_hbm, i_hbm, x_hbm, emb_hbm,
             sidx, iidx, bufS, bufI, accb, semS, semI):
        wid = lax.axis_index("s") * _NC + lax.axis_index("c")
        tile_base = wid * chunk

        def step_body(step, carry):
            base = tile_base + step * SUB
            pltpu.sync_copy(s_hbm.at[pl.ds(base, SUB)], sidx)
            pltpu.sync_copy(i_hbm.at[pl.ds(base, SUB)], iidx)
            cS = pltpu.async_copy(x_hbm.at[sidx], bufS, semS)
            cI = pltpu.async_copy(x_hbm.at[iidx], bufI, semI)
            cS.wait()
            cI.wait()

            def acc_body(r, carry2):
                for g in range(D // 16):
                    sl = pl.ds(g * 16, 16)
                    sl_hi = pl.ds(D + g * 16, 16)
                    accb[r, sl] = bufS[r, sl] + bufI[r, sl_hi]
                return carry2
            lax.fori_loop(0, SUB, acc_body, 0)
            pltpu.sync_copy(accb, emb_hbm.at[pl.ds(base, SUB)])
            return carry

        lax.fori_loop(0, nstep, step_body, 0)

    return sc_k(s_tok, i_tok, X)


def _sc_gather_y(a_tok, m_flat, Y):
    B = a_tok.shape[0]
    W2 = Y.shape[1]
    D = W2 // 2
    f32 = jnp.float32
    SUB = 128
    chunk = B // _NW
    nstep = chunk // SUB

    @functools.partial(
        pl.kernel,
        out_type=(
            jax.ShapeDtypeStruct((B, D), f32),
            jax.ShapeDtypeStruct((B, D), f32),
        ),
        mesh=_mesh(),
        compiler_params=pltpu.CompilerParams(use_tc_tiling_on_sc=True),
        scratch_types=[
            pltpu.VMEM((SUB,), jnp.int32),
            pltpu.VMEM((4, SUB), jnp.int32),
            pltpu.VMEM((SUB, W2), f32),
            pltpu.VMEM((4 * SUB, W2), f32),
            pltpu.VMEM((SUB, D), f32),
            pltpu.VMEM((SUB, D), f32),
            pltpu.SemaphoreType.DMA,
            pltpu.SemaphoreType.DMA,
        ],
    )
    def sc_k(a_hbm, m_hbm, y_hbm, embA_hbm, msum_hbm,
             aidx, midx, bufA, mrows, accb, msumb, semA, semM):
        wid = lax.axis_index("s") * _NC + lax.axis_index("c")
        tile_base = wid * chunk

        def step_body(step, carry):
            base = tile_base + step * SUB
            pltpu.sync_copy(a_hbm.at[pl.ds(base, SUB)], aidx)
            for j in range(4):
                pltpu.sync_copy(m_hbm.at[pl.ds(4 * base + j * SUB, SUB)],
                                midx.at[j])
            cA = pltpu.async_copy(y_hbm.at[aidx], bufA, semA)
            cM = [pltpu.async_copy(y_hbm.at[midx.at[j]],
                                   mrows.at[pl.ds(j * SUB, SUB)], semM)
                  for j in range(4)]
            cA.wait()

            def acc_body(r, carry2):
                for g in range(D // 16):
                    sl = pl.ds(g * 16, 16)
                    accb[r, sl] = bufA[r, sl] + 0.0
                return carry2
            lax.fori_loop(0, SUB, acc_body, 0)
            pltpu.sync_copy(accb, embA_hbm.at[pl.ds(base, SUB)])

            for c in cM:
                c.wait()

            def msum_body(t, carry2):
                for g in range(D // 16):
                    sl = pl.ds(g * 16, 16)
                    sl_hi = pl.ds(D + g * 16, 16)
                    msumb[t, sl] = (mrows[4 * t, sl_hi]
                                    + mrows[4 * t + 1, sl_hi]
                                    + mrows[4 * t + 2, sl_hi]
                                    + mrows[4 * t + 3, sl_hi])
                return carry2
            lax.fori_loop(0, SUB, msum_body, 0)
            pltpu.sync_copy(msumb, msum_hbm.at[pl.ds(base, SUB)])
            return carry

        lax.fori_loop(0, nstep, step_body, 0)

    return sc_k(a_tok, m_flat, Y)


def _tc_combine(embSI, embA, msum, aux8,
                Ws, Wi, Wa, Wm, r0s, r0i, r0a, r0m):
    B, D = embSI.shape
    P = r0s.shape[1]
    BLK = min(2048, B)
    f32 = jnp.float32

    def body(e1_r, e2_r, msum_r, aux_r,
             ws_r, wi_r, wa_r, wm_r, r0s_r, r0i_r, r0a_r, r0m_r, out_r):
        cs = jnp.dot(r0s_r[...], ws_r[...], preferred_element_type=f32)
        ci = jnp.dot(r0i_r[...], wi_r[...], preferred_element_type=f32)
        ca = jnp.dot(r0a_r[...], wa_r[...], preferred_element_type=f32)
        cm = jnp.dot(r0m_r[...], wm_r[...], preferred_element_type=f32)
        aux = aux_r[...]
        zs = (aux[:, 0:1] == 0).astype(f32)
        zi = (aux[:, 1:2] == 0).astype(f32)
        za = (aux[:, 2:3] == 0).astype(f32)
        nmf = jnp.maximum(aux[:, 3:4], 1).astype(f32)
        cnt0 = jnp.sum((aux[:, 4:8] == 0).astype(f32), axis=1, keepdims=True)
        out_r[...] = (e1_r[...] + e2_r[...] + (msum_r[...] - cnt0 * cm) / nmf
                      - zs * cs - zi * ci - za * ca)

    blk_bd = pl.BlockSpec((BLK, D), lambda i: (i, 0))
    blk_b8 = pl.BlockSpec((BLK, 8), lambda i: (i, 0))
    blk_w = pl.BlockSpec((P, D), lambda i: (0, 0))
    blk_r0 = pl.BlockSpec((1, P), lambda i: (0, 0))

    return pl.pallas_call(
        body,
        grid=(B // BLK,),
        in_specs=[blk_bd, blk_bd, blk_bd, blk_b8,
                  blk_w, blk_w, blk_w, blk_w,
                  blk_r0, blk_r0, blk_r0, blk_r0],
        out_specs=blk_bd,
        out_shape=jax.ShapeDtypeStruct((B, D), f32),
    )(embSI, embA, msum, aux8, Ws, Wi, Wa, Wm, r0s, r0i, r0a, r0m)


def kernel(species_tokens, ability_tokens, item_tokens, move_tokens, num_moves,
           species_table, items_table, abilities_table, moves_table,
           pre_species, pre_items, pre_abilities, pre_moves,
           species_W, items_W, abilities_W, moves_W):
    m_flat = move_tokens.reshape(-1)
    aux8 = jnp.concatenate(
        [species_tokens[:, None], item_tokens[:, None],
         ability_tokens[:, None], num_moves[:, None], move_tokens], axis=1)
    X = _tc_fold_pair(species_table.T, items_table.T,
                      pre_species, pre_items, species_W, items_W)
    Y = _tc_fold_pair(abilities_table.T, moves_table.T,
                      pre_abilities, pre_moves, abilities_W, moves_W)
    embSI = _sc_gather_x(species_tokens, item_tokens, X)
    embA, msum = _sc_gather_y(ability_tokens, m_flat, Y)
    return _tc_combine(
        embSI, embA, msum, aux8,
        species_W, items_W, abilities_W, moves_W,
        pre_species[0:1], pre_items[0:1], pre_abilities[0:1], pre_moves[0:1])


# speedup vs baseline: 1.0595x; 1.0595x over previous
import functools

import jax
import jax.numpy as jnp
from jax import lax
from jax.experimental import pallas as pl
from jax.experimental.pallas import tpu as pltpu
from jax.experimental.pallas import tpu_sc as plsc

_NC, _NS = 2, 16
_NW = _NC * _NS
_VB = 2048


def _tc_fold_pair(lt1, lt2, pre1, pre2, W1, W2, l0pair):
    D, V = lt1.shape
    P = pre1.shape[1]
    grid_n = (V + _VB - 1) // _VB + (1 if V % _VB == 0 else 0)
    special = V
    VPAD = grid_n * _VB
    f32 = jnp.float32

    def body(lt1_r, lt2_r, p1_r, p2_r, w1_r, w2_r, l0_r, out_r):
        def fused(lt_r, p_r, w_r):
            lt = jnp.transpose(lt_r[...], (1, 0))
            return lt + jnp.dot(p_r[...], w_r[...], preferred_element_type=f32)
        full = jnp.concatenate(
            [fused(lt1_r, p1_r, w1_r), fused(lt2_r, p2_r, w2_r)], axis=1)
        rows = (pl.program_id(0) * _VB
                + jax.lax.broadcasted_iota(jnp.int32, (_VB, 1), 0))
        out_r[...] = jnp.where(rows == special, l0_r[...], full)

    blk_lt = pl.BlockSpec((D, _VB), lambda i: (0, i))
    blk_p = pl.BlockSpec((_VB, P), lambda i: (i, 0))
    blk_w = pl.BlockSpec((P, D), lambda i: (0, 0))
    blk_l0 = pl.BlockSpec((1, 2 * D), lambda i: (0, 0))
    blk_o = pl.BlockSpec((_VB, 2 * D), lambda i: (i, 0))

    return pl.pallas_call(
        body,
        grid=(grid_n,),
        in_specs=[blk_lt, blk_lt, blk_p, blk_p, blk_w, blk_w, blk_l0],
        out_specs=blk_o,
        out_shape=jax.ShapeDtypeStruct((VPAD, 2 * D), f32),
    )(lt1, lt2, pre1, pre2, W1, W2, l0pair)


def _mesh():
    return plsc.VectorSubcoreMesh(core_axis_name="c", subcore_axis_name="s",
                                  num_cores=_NC, num_subcores=_NS)


def _redirect_zeros(idx_ref, n, special):
    for g in range(n // 16):
        sl = pl.ds(g * 16, 16)
        v = idx_ref[sl]
        idx_ref[sl] = jnp.where(v == 0, special, v)


def _sc_gather_y(a_tok, m_flat, num_moves, Y, special):
    B = a_tok.shape[0]
    W2 = Y.shape[1]
    D = W2 // 2
    f32 = jnp.float32
    SUB = 128
    chunk = B // _NW
    nstep = chunk // SUB

    @functools.partial(
        pl.kernel,
        out_type=jax.ShapeDtypeStruct((B, D), f32),
        mesh=_mesh(),
        compiler_params=pltpu.CompilerParams(use_tc_tiling_on_sc=True),
        scratch_types=[
            pltpu.VMEM((SUB,), jnp.int32),
            pltpu.VMEM((4, SUB), jnp.int32),
            pltpu.VMEM((SUB + 16,), jnp.int32),
            pltpu.VMEM((SUB, W2), f32),
            pltpu.VMEM((4 * SUB, W2), f32),
            pltpu.VMEM((SUB, D), f32),
            pltpu.SemaphoreType.DMA,
            pltpu.SemaphoreType.DMA,
        ],
    )
    def sc_k(a_hbm, m_hbm, nm_hbm, y_hbm, out_hbm,
             aidx, midx, nmbuf, bufA, mrows, outb, semA, semM):
        wid = lax.axis_index("s") * _NC + lax.axis_index("c")
        tile_base = wid * chunk

        def step_body(step, carry):
            base = tile_base + step * SUB
            pltpu.sync_copy(a_hbm.at[pl.ds(base, SUB)], aidx)
            pltpu.sync_copy(nm_hbm.at[pl.ds(base, SUB)], nmbuf.at[pl.ds(0, SUB)])
            for j in range(4):
                pltpu.sync_copy(m_hbm.at[pl.ds(4 * base + j * SUB, SUB)],
                                midx.at[j])
            _redirect_zeros(aidx, SUB, special)
            for j in range(4):
                for g in range(SUB // 16):
                    sl = pl.ds(g * 16, 16)
                    v = midx[j, sl]
                    midx[j, sl] = jnp.where(v == 0, special, v)

            cA = pltpu.async_copy(y_hbm.at[aidx], bufA, semA)
            cM = [pltpu.async_copy(y_hbm.at[midx.at[j]],
                                   mrows.at[pl.ds(j * SUB, SUB)], semM)
                  for j in range(4)]
            cA.wait()
            for c in cM:
                c.wait()

            third = jnp.float32(1.0 / 3.0)

            def row_body(t, carry2):
                nm = nmbuf[pl.ds(t, 16)][0]
                q = jnp.where(
                    nm <= 1, jnp.float32(1.0),
                    jnp.where(nm == 2, jnp.float32(0.5),
                              jnp.where(nm == 3, third, jnp.float32(0.25))))
                for g in range(D // 16):
                    sl = pl.ds(g * 16, 16)
                    sl_hi = pl.ds(D + g * 16, 16)
                    msum = (mrows[4 * t, sl_hi] + mrows[4 * t + 1, sl_hi]
                            + mrows[4 * t + 2, sl_hi]
                            + mrows[4 * t + 3, sl_hi])
                    outb[t, sl] = bufA[t, sl] + msum * q
                return carry2
            lax.fori_loop(0, SUB, row_body, 0)
            pltpu.sync_copy(outb, out_hbm.at[pl.ds(base, SUB)])
            return carry

        lax.fori_loop(0, nstep, step_body, 0)

    return sc_k(a_tok, m_flat, num_moves, Y)


def _sc_gather_x(s_tok, i_tok, partialY, X, special):
    B = s_tok.shape[0]
    W2 = X.shape[1]
    D = W2 // 2
    f32 = jnp.float32
    SUB = 128
    chunk = B // _NW
    nstep = chunk // SUB

    @functools.partial(
        pl.kernel,
        out_type=jax.ShapeDtypeStruct((B, D), f32),
        mesh=_mesh(),
        compiler_params=pltpu.CompilerParams(use_tc_tiling_on_sc=True),
        scratch_types=[
            pltpu.VMEM((SUB,), jnp.int32),
            pltpu.VMEM((SUB,), jnp.int32),
            pltpu.VMEM((SUB, W2), f32),
            pltpu.VMEM((SUB, W2), f32),
            pltpu.VMEM((SUB, D), f32),
            pltpu.VMEM((SUB, D), f32),
            pltpu.SemaphoreType.DMA,
            pltpu.SemaphoreType.DMA,
        ],
    )
    def sc_k(s_hbm, i_hbm, p_hbm, x_hbm, out_hbm,
             sidx, iidx, bufS, bufI, pbuf, outb, semS, semI):
        wid = lax.axis_index("s") * _NC + lax.axis_index("c")
        tile_base = wid * chunk

        def step_body(step, carry):
            base = tile_base + step * SUB
            pltpu.sync_copy(s_hbm.at[pl.ds(base, SUB)], sidx)
            pltpu.sync_copy(i_hbm.at[pl.ds(base, SUB)], iidx)
            _redirect_zeros(sidx, SUB, special)
            _redirect_zeros(iidx, SUB, special)
            cS = pltpu.async_copy(x_hbm.at[sidx], bufS, semS)
            cI = pltpu.async_copy(x_hbm.at[iidx], bufI, semI)
            pltpu.sync_copy(p_hbm.at[pl.ds(base, SUB)], pbuf)
            cS.wait()
            cI.wait()

            def row_body(r, carry2):
                for g in range(D // 16):
                    sl = pl.ds(g * 16, 16)
                    sl_hi = pl.ds(D + g * 16, 16)
                    outb[r, sl] = bufS[r, sl] + bufI[r, sl_hi] + pbuf[r, sl]
                return carry2
            lax.fori_loop(0, SUB, row_body, 0)
            pltpu.sync_copy(outb, out_hbm.at[pl.ds(base, SUB)])
            return carry

        lax.fori_loop(0, nstep, step_body, 0)

    return sc_k(s_tok, i_tok, partialY, X)


def kernel(species_tokens, ability_tokens, item_tokens, move_tokens, num_moves,
           species_table, items_table, abilities_table, moves_table,
           pre_species, pre_items, pre_abilities, pre_moves,
           species_W, items_W, abilities_W, moves_W):
    m_flat = move_tokens.reshape(-1)
    l0_x = jnp.concatenate([species_table[0:1], items_table[0:1]], axis=1)
    l0_y = jnp.concatenate([abilities_table[0:1], moves_table[0:1]], axis=1)
    Y = _tc_fold_pair(abilities_table.T, moves_table.T,
                      pre_abilities, pre_moves, abilities_W, moves_W, l0_y)
    X = _tc_fold_pair(species_table.T, items_table.T,
                      pre_species, pre_items, species_W, items_W, l0_x)
    V = species_table.shape[0]
    partialY = _sc_gather_y(ability_tokens, m_flat, num_moves, Y, V)
    return _sc_gather_x(species_tokens, item_tokens, partialY, X, V)


# speedup vs baseline: 1.1261x; 1.0629x over previous
import functools

import jax
import jax.numpy as jnp
from jax import lax
from jax.experimental import pallas as pl
from jax.experimental.pallas import tpu as pltpu
from jax.experimental.pallas import tpu_sc as plsc

_NC, _NS = 2, 16
_NW = _NC * _NS
_VB = 2048


def _tc_fold_pair(lt1, lt2, pre1, pre2, W1, W2, l0pair):
    D, V = lt1.shape
    P = pre1.shape[1]
    grid_n = (V + _VB - 1) // _VB + (1 if V % _VB == 0 else 0)
    special = V
    VPAD = grid_n * _VB
    f32 = jnp.float32

    def body(lt1_r, lt2_r, p1_r, p2_r, w1_r, w2_r, l0_r, out_r):
        def fused(lt_r, p_r, w_r):
            lt = jnp.transpose(lt_r[...], (1, 0))
            return lt + jnp.dot(p_r[...], w_r[...], preferred_element_type=f32)
        full = jnp.concatenate(
            [fused(lt1_r, p1_r, w1_r), fused(lt2_r, p2_r, w2_r)], axis=1)
        rows = (pl.program_id(0) * _VB
                + jax.lax.broadcasted_iota(jnp.int32, (_VB, 1), 0))
        out_r[...] = jnp.where(rows == special, l0_r[...], full)

    blk_lt = pl.BlockSpec((D, _VB), lambda i: (0, i))
    blk_p = pl.BlockSpec((_VB, P), lambda i: (i, 0))
    blk_w = pl.BlockSpec((P, D), lambda i: (0, 0))
    blk_l0 = pl.BlockSpec((1, 2 * D), lambda i: (0, 0))
    blk_o = pl.BlockSpec((_VB, 2 * D), lambda i: (i, 0))

    return pl.pallas_call(
        body,
        grid=(grid_n,),
        in_specs=[blk_lt, blk_lt, blk_p, blk_p, blk_w, blk_w, blk_l0],
        out_specs=blk_o,
        out_shape=jax.ShapeDtypeStruct((VPAD, 2 * D), f32),
    )(lt1, lt2, pre1, pre2, W1, W2, l0pair)


def _mesh():
    return plsc.VectorSubcoreMesh(core_axis_name="c", subcore_axis_name="s",
                                  num_cores=_NC, num_subcores=_NS)


def _redirect_zeros(idx_ref, n, special):
    for g in range(n // 16):
        sl = pl.ds(g * 16, 16)
        v = idx_ref[sl]
        idx_ref[sl] = jnp.where(v == 0, special, v)


def _sc_gather_y(a_tok, m_flat, num_moves, Y, special):
    B = a_tok.shape[0]
    W2 = Y.shape[1]
    D = W2 // 2
    f32 = jnp.float32
    SUB = 128
    chunk = B // _NW
    nstep = chunk // SUB

    @functools.partial(
        pl.kernel,
        out_type=jax.ShapeDtypeStruct((B, D), f32),
        mesh=_mesh(),
        compiler_params=pltpu.CompilerParams(use_tc_tiling_on_sc=True),
        scratch_types=[
            pltpu.VMEM((SUB,), jnp.int32),
            pltpu.VMEM((4, SUB), jnp.int32),
            pltpu.VMEM((SUB + 16,), jnp.int32),
            pltpu.VMEM((SUB, W2), f32),
            pltpu.VMEM((4 * SUB, W2), f32),
            pltpu.VMEM((SUB, D), f32),
            pltpu.SemaphoreType.DMA,
            pltpu.SemaphoreType.DMA,
        ],
    )
    def sc_k(a_hbm, m_hbm, nm_hbm, y_hbm, out_hbm,
             aidx, midx, nmbuf, bufA, mrows, outb, semA, semM):
        wid = lax.axis_index("s") * _NC + lax.axis_index("c")
        tile_base = wid * chunk

        def step_body(step, carry):
            base = tile_base + step * SUB
            pltpu.sync_copy(a_hbm.at[pl.ds(base, SUB)], aidx)
            pltpu.sync_copy(nm_hbm.at[pl.ds(base, SUB)], nmbuf.at[pl.ds(0, SUB)])
            for j in range(4):
                pltpu.sync_copy(m_hbm.at[j, pl.ds(base, SUB)], midx.at[j])
            _redirect_zeros(aidx, SUB, special)
            for j in range(4):
                for g in range(SUB // 16):
                    sl = pl.ds(g * 16, 16)
                    v = midx[j, sl]
                    midx[j, sl] = jnp.where(v == 0, special, v)

            cA = pltpu.async_copy(y_hbm.at[aidx], bufA, semA)
            cM = [pltpu.async_copy(y_hbm.at[midx.at[j]],
                                   mrows.at[pl.ds(j * SUB, SUB)], semM)
                  for j in range(4)]
            cA.wait()
            for c in cM:
                c.wait()

            third = jnp.float32(1.0 / 3.0)

            def row_body(t, carry2):
                nm = nmbuf[pl.ds(t, 16)][0]
                q = jnp.where(
                    nm <= 1, jnp.float32(1.0),
                    jnp.where(nm == 2, jnp.float32(0.5),
                              jnp.where(nm == 3, third, jnp.float32(0.25))))
                for g in range(D // 16):
                    sl = pl.ds(g * 16, 16)
                    sl_hi = pl.ds(D + g * 16, 16)
                    msum = (mrows[t, sl_hi] + mrows[SUB + t, sl_hi]
                            + mrows[2 * SUB + t, sl_hi]
                            + mrows[3 * SUB + t, sl_hi])
                    outb[t, sl] = bufA[t, sl] + msum * q
                return carry2
            lax.fori_loop(0, SUB, row_body, 0)
            pltpu.sync_copy(outb, out_hbm.at[pl.ds(base, SUB)])
            return carry

        lax.fori_loop(0, nstep, step_body, 0)

    return sc_k(a_tok, m_flat, num_moves, Y)


def _sc_gather_x(s_tok, i_tok, partialY, X, special):
    B = s_tok.shape[0]
    W2 = X.shape[1]
    D = W2 // 2
    f32 = jnp.float32
    SUB = 128
    chunk = B // _NW
    nstep = chunk // SUB

    @functools.partial(
        pl.kernel,
        out_type=jax.ShapeDtypeStruct((B, D), f32),
        mesh=_mesh(),
        compiler_params=pltpu.CompilerParams(use_tc_tiling_on_sc=True),
        scratch_types=[
            pltpu.VMEM((2, SUB), jnp.int32),
            pltpu.VMEM((2, SUB), jnp.int32),
            pltpu.VMEM((2 * SUB, W2), f32),
            pltpu.VMEM((2 * SUB, W2), f32),
            pltpu.VMEM((SUB, D), f32),
            pltpu.VMEM((SUB, D), f32),
            pltpu.SemaphoreType.DMA,
            pltpu.SemaphoreType.DMA,
            pltpu.SemaphoreType.DMA,
            pltpu.SemaphoreType.DMA,
        ],
    )
    def sc_k(s_hbm, i_hbm, p_hbm, x_hbm, out_hbm,
             sidx, iidx, bufS, bufI, pbuf, outb, semS0, semI0, semS1, semI1):
        wid = lax.axis_index("s") * _NC + lax.axis_index("c")
        tile_base = wid * chunk

        sems = [(semS0, semI0), (semS1, semI1)]

        def fire(step):
            par = step % 2
            base = tile_base + step * SUB
            semS, semI = sems[par]
            pltpu.sync_copy(s_hbm.at[pl.ds(base, SUB)], sidx.at[par])
            pltpu.sync_copy(i_hbm.at[pl.ds(base, SUB)], iidx.at[par])
            for g in range(SUB // 16):
                sl = pl.ds(g * 16, 16)
                v = sidx[par, sl]
                sidx[par, sl] = jnp.where(v == 0, special, v)
                w = iidx[par, sl]
                iidx[par, sl] = jnp.where(w == 0, special, w)
            half = pl.ds(par * SUB, SUB)
            cS = pltpu.async_copy(x_hbm.at[sidx.at[par]], bufS.at[half], semS)
            cI = pltpu.async_copy(x_hbm.at[iidx.at[par]], bufI.at[half], semI)
            return cS, cI

        def drain(step, cS, cI):
            par = step % 2
            base = tile_base + step * SUB
            pltpu.sync_copy(p_hbm.at[pl.ds(base, SUB)], pbuf)
            cS.wait()
            cI.wait()
            off = par * SUB

            def row_body(r, carry2):
                for g in range(D // 16):
                    sl = pl.ds(g * 16, 16)
                    sl_hi = pl.ds(D + g * 16, 16)
                    outb[r, sl] = (bufS[off + r, sl] + bufI[off + r, sl_hi]
                                   + pbuf[r, sl])
                return carry2
            lax.fori_loop(0, SUB, row_body, 0)
            pltpu.sync_copy(outb, out_hbm.at[pl.ds(base, SUB)])

        pending = fire(0)
        for step in range(nstep):
            nxt = fire(step + 1) if step + 1 < nstep else None
            drain(step, *pending)
            pending = nxt

    return sc_k(s_tok, i_tok, partialY, X)


def kernel(species_tokens, ability_tokens, item_tokens, move_tokens, num_moves,
           species_table, items_table, abilities_table, moves_table,
           pre_species, pre_items, pre_abilities, pre_moves,
           species_W, items_W, abilities_W, moves_W):
    m_t = move_tokens.T
    l0_x = jnp.concatenate([species_table[0:1], items_table[0:1]], axis=1)
    l0_y = jnp.concatenate([abilities_table[0:1], moves_table[0:1]], axis=1)
    Y = _tc_fold_pair(abilities_table.T, moves_table.T,
                      pre_abilities, pre_moves, abilities_W, moves_W, l0_y)
    X = _tc_fold_pair(species_table.T, items_table.T,
                      pre_species, pre_items, species_W, items_W, l0_x)
    V = species_table.shape[0]
    partialY = _sc_gather_y(ability_tokens, m_t, num_moves, Y, V)
    return _sc_gather_x(species_tokens, item_tokens, partialY, X, V)


# speedup vs baseline: 1.2500x; 1.1100x over previous
import functools

import jax
import jax.numpy as jnp
from jax import lax
from jax.experimental import pallas as pl
from jax.experimental.pallas import tpu as pltpu
from jax.experimental.pallas import tpu_sc as plsc

_NC, _NS = 2, 16
_NW = _NC * _NS
_VB = 4096


def _tc_fold_pair(lt1, lt2, pre1, pre2, W1, W2, l0pair):
    D, V = lt1.shape
    P = pre1.shape[1]
    grid_n = (V + _VB - 1) // _VB + (1 if V % _VB == 0 else 0)
    special = V
    VPAD = grid_n * _VB
    f32 = jnp.float32

    def body(lt1_r, lt2_r, p1_r, p2_r, w1_r, w2_r, l0_r, out_r):
        def fused(lt_r, p_r, w_r):
            lt = jnp.transpose(lt_r[...], (1, 0))
            return lt + jnp.dot(p_r[...], w_r[...], preferred_element_type=f32)
        full = jnp.concatenate(
            [fused(lt1_r, p1_r, w1_r), fused(lt2_r, p2_r, w2_r)], axis=1)
        rows = (pl.program_id(0) * _VB
                + jax.lax.broadcasted_iota(jnp.int32, (_VB, 1), 0))
        out_r[...] = jnp.where(rows == special, l0_r[...], full)

    blk_lt = pl.BlockSpec((D, _VB), lambda i: (0, i))
    blk_p = pl.BlockSpec((_VB, P), lambda i: (i, 0))
    blk_w = pl.BlockSpec((P, D), lambda i: (0, 0))
    blk_l0 = pl.BlockSpec((1, 2 * D), lambda i: (0, 0))
    blk_o = pl.BlockSpec((_VB, 2 * D), lambda i: (i, 0))

    return pl.pallas_call(
        body,
        grid=(grid_n,),
        in_specs=[blk_lt, blk_lt, blk_p, blk_p, blk_w, blk_w, blk_l0],
        out_specs=blk_o,
        out_shape=jax.ShapeDtypeStruct((VPAD, 2 * D), f32),
    )(lt1, lt2, pre1, pre2, W1, W2, l0pair)


def _mesh():
    return plsc.VectorSubcoreMesh(core_axis_name="c", subcore_axis_name="s",
                                  num_cores=_NC, num_subcores=_NS)


def _redirect_zeros(idx_ref, n, special):
    for g in range(n // 16):
        sl = pl.ds(g * 16, 16)
        v = idx_ref[sl]
        idx_ref[sl] = jnp.where(v == 0, special, v)


def _sc_gather_y(a_tok, m_flat, num_moves, Y, special):
    B = a_tok.shape[0]
    W2 = Y.shape[1]
    D = W2 // 2
    f32 = jnp.float32
    SUB = 128
    chunk = B // _NW
    nstep = chunk // SUB

    @functools.partial(
        pl.kernel,
        out_type=jax.ShapeDtypeStruct((B, D), f32),
        mesh=_mesh(),
        compiler_params=pltpu.CompilerParams(use_tc_tiling_on_sc=True),
        scratch_types=[
            pltpu.VMEM((SUB,), jnp.int32),
            pltpu.VMEM((4, SUB), jnp.int32),
            pltpu.VMEM((SUB + 16,), jnp.int32),
            pltpu.VMEM((SUB, W2), f32),
            pltpu.VMEM((4 * SUB, W2), f32),
            pltpu.VMEM((SUB, D), f32),
            pltpu.SemaphoreType.DMA,
            pltpu.SemaphoreType.DMA,
        ],
    )
    def sc_k(a_hbm, m_hbm, nm_hbm, y_hbm, out_hbm,
             aidx, midx, nmbuf, bufA, mrows, outb, semA, semM):
        wid = lax.axis_index("s") * _NC + lax.axis_index("c")
        tile_base = wid * chunk

        def step_body(step, carry):
            base = tile_base + step * SUB
            pltpu.sync_copy(a_hbm.at[pl.ds(base, SUB)], aidx)
            pltpu.sync_copy(nm_hbm.at[pl.ds(base, SUB)], nmbuf.at[pl.ds(0, SUB)])
            for j in range(4):
                pltpu.sync_copy(m_hbm.at[j, pl.ds(base, SUB)], midx.at[j])
            _redirect_zeros(aidx, SUB, special)
            for j in range(4):
                for g in range(SUB // 16):
                    sl = pl.ds(g * 16, 16)
                    v = midx[j, sl]
                    midx[j, sl] = jnp.where(v == 0, special, v)

            cA = pltpu.async_copy(y_hbm.at[aidx], bufA, semA)
            cM = [pltpu.async_copy(y_hbm.at[midx.at[j]],
                                   mrows.at[pl.ds(j * SUB, SUB)], semM)
                  for j in range(4)]
            cA.wait()
            for c in cM:
                c.wait()

            third = jnp.float32(1.0 / 3.0)

            def row_body(t, carry2):
                nm = nmbuf[pl.ds(t, 16)][0]
                q = jnp.where(
                    nm <= 1, jnp.float32(1.0),
                    jnp.where(nm == 2, jnp.float32(0.5),
                              jnp.where(nm == 3, third, jnp.float32(0.25))))
                for g in range(D // 16):
                    sl = pl.ds(g * 16, 16)
                    sl_hi = pl.ds(D + g * 16, 16)
                    msum = (mrows[t, sl_hi] + mrows[SUB + t, sl_hi]
                            + mrows[2 * SUB + t, sl_hi]
                            + mrows[3 * SUB + t, sl_hi])
                    outb[t, sl] = bufA[t, sl] + msum * q
                return carry2
            lax.fori_loop(0, SUB, row_body, 0)
            pltpu.sync_copy(outb, out_hbm.at[pl.ds(base, SUB)])
            return carry

        lax.fori_loop(0, nstep, step_body, 0)

    return sc_k(a_tok, m_flat, num_moves, Y)


def _sc_gather_x(s_tok, i_tok, partialY, X, special):
    B = s_tok.shape[0]
    W2 = X.shape[1]
    D = W2 // 2
    f32 = jnp.float32
    SUB = 128
    chunk = B // _NW
    nstep = chunk // SUB

    @functools.partial(
        pl.kernel,
        out_type=jax.ShapeDtypeStruct((B, D), f32),
        mesh=_mesh(),
        compiler_params=pltpu.CompilerParams(use_tc_tiling_on_sc=True),
        scratch_types=[
            pltpu.VMEM((2, SUB), jnp.int32),
            pltpu.VMEM((2, SUB), jnp.int32),
            pltpu.VMEM((2 * SUB, W2), f32),
            pltpu.VMEM((2 * SUB, W2), f32),
            pltpu.VMEM((SUB, D), f32),
            pltpu.VMEM((SUB, D), f32),
            pltpu.SemaphoreType.DMA,
            pltpu.SemaphoreType.DMA,
            pltpu.SemaphoreType.DMA,
            pltpu.SemaphoreType.DMA,
        ],
    )
    def sc_k(s_hbm, i_hbm, p_hbm, x_hbm, out_hbm,
             sidx, iidx, bufS, bufI, pbuf, outb, semS0, semI0, semS1, semI1):
        wid = lax.axis_index("s") * _NC + lax.axis_index("c")
        tile_base = wid * chunk

        sems = [(semS0, semI0), (semS1, semI1)]

        def fire(step):
            par = step % 2
            base = tile_base + step * SUB
            semS, semI = sems[par]
            pltpu.sync_copy(s_hbm.at[pl.ds(base, SUB)], sidx.at[par])
            pltpu.sync_copy(i_hbm.at[pl.ds(base, SUB)], iidx.at[par])
            for g in range(SUB // 16):
                sl = pl.ds(g * 16, 16)
                v = sidx[par, sl]
                sidx[par, sl] = jnp.where(v == 0, special, v)
                w = iidx[par, sl]
                iidx[par, sl] = jnp.where(w == 0, special, w)
            half = pl.ds(par * SUB, SUB)
            cS = pltpu.async_copy(x_hbm.at[sidx.at[par]], bufS.at[half], semS)
            cI = pltpu.async_copy(x_hbm.at[iidx.at[par]], bufI.at[half], semI)
            return cS, cI

        def drain(step, cS, cI):
            par = step % 2
            base = tile_base + step * SUB
            pltpu.sync_copy(p_hbm.at[pl.ds(base, SUB)], pbuf)
            cS.wait()
            cI.wait()
            off = par * SUB

            def row_body(r, carry2):
                for g in range(D // 16):
                    sl = pl.ds(g * 16, 16)
                    sl_hi = pl.ds(D + g * 16, 16)
                    outb[r, sl] = (bufS[off + r, sl] + bufI[off + r, sl_hi]
                                   + pbuf[r, sl])
                return carry2
            lax.fori_loop(0, SUB, row_body, 0)
            pltpu.sync_copy(outb, out_hbm.at[pl.ds(base, SUB)])

        pending = fire(0)
        for step in range(nstep):
            nxt = fire(step + 1) if step + 1 < nstep else None
            drain(step, *pending)
            pending = nxt

    return sc_k(s_tok, i_tok, partialY, X)


def kernel(species_tokens, ability_tokens, item_tokens, move_tokens, num_moves,
           species_table, items_table, abilities_table, moves_table,
           pre_species, pre_items, pre_abilities, pre_moves,
           species_W, items_W, abilities_W, moves_W):
    m_t = move_tokens.T
    l0_x = jnp.concatenate([species_table[0:1], items_table[0:1]], axis=1)
    l0_y = jnp.concatenate([abilities_table[0:1], moves_table[0:1]], axis=1)
    Y = _tc_fold_pair(abilities_table.T, moves_table.T,
                      pre_abilities, pre_moves, abilities_W, moves_W, l0_y)
    X = _tc_fold_pair(species_table.T, items_table.T,
                      pre_species, pre_items, species_W, items_W, l0_x)
    V = species_table.shape[0]
    partialY = _sc_gather_y(ability_tokens, m_t, num_moves, Y, V)
    return _sc_gather_x(species_tokens, item_tokens, partialY, X, V)


# speedup vs baseline: 1.2578x; 1.0063x over previous
import functools

import jax
import jax.numpy as jnp
from jax import lax
from jax.experimental import pallas as pl
from jax.experimental.pallas import tpu as pltpu
from jax.experimental.pallas import tpu_sc as plsc

_NC, _NS = 2, 16
_NW = _NC * _NS
_VB = 8192


def _tc_fold_pair(lt1, lt2, pre1, pre2, W1, W2, l0pair):
    D, V = lt1.shape
    P = pre1.shape[1]
    grid_n = (V + _VB - 1) // _VB + (1 if V % _VB == 0 else 0)
    special = V
    VPAD = grid_n * _VB
    f32 = jnp.float32

    def body(lt1_r, lt2_r, p1_r, p2_r, w1_r, w2_r, l0_r, out_r):
        def fused(lt_r, p_r, w_r):
            lt = jnp.transpose(lt_r[...], (1, 0))
            return lt + jnp.dot(p_r[...], w_r[...], preferred_element_type=f32)
        full = jnp.concatenate(
            [fused(lt1_r, p1_r, w1_r), fused(lt2_r, p2_r, w2_r)], axis=1)
        rows = (pl.program_id(0) * _VB
                + jax.lax.broadcasted_iota(jnp.int32, (_VB, 1), 0))
        out_r[...] = jnp.where(rows == special, l0_r[...], full)

    blk_lt = pl.BlockSpec((D, _VB), lambda i: (0, i))
    blk_p = pl.BlockSpec((_VB, P), lambda i: (i, 0))
    blk_w = pl.BlockSpec((P, D), lambda i: (0, 0))
    blk_l0 = pl.BlockSpec((1, 2 * D), lambda i: (0, 0))
    blk_o = pl.BlockSpec((_VB, 2 * D), lambda i: (i, 0))

    return pl.pallas_call(
        body,
        grid=(grid_n,),
        in_specs=[blk_lt, blk_lt, blk_p, blk_p, blk_w, blk_w, blk_l0],
        out_specs=blk_o,
        out_shape=jax.ShapeDtypeStruct((VPAD, 2 * D), f32),
    )(lt1, lt2, pre1, pre2, W1, W2, l0pair)


def _mesh():
    return plsc.VectorSubcoreMesh(core_axis_name="c", subcore_axis_name="s",
                                  num_cores=_NC, num_subcores=_NS)


def _redirect_zeros(idx_ref, n, special):
    for g in range(n // 16):
        sl = pl.ds(g * 16, 16)
        v = idx_ref[sl]
        idx_ref[sl] = jnp.where(v == 0, special, v)


def _sc_gather_y(a_tok, m_flat, num_moves, Y, special):
    B = a_tok.shape[0]
    W2 = Y.shape[1]
    D = W2 // 2
    f32 = jnp.float32
    SUB = 128
    chunk = B // _NW
    nstep = chunk // SUB

    @functools.partial(
        pl.kernel,
        out_type=jax.ShapeDtypeStruct((B, D), f32),
        mesh=_mesh(),
        compiler_params=pltpu.CompilerParams(use_tc_tiling_on_sc=True),
        scratch_types=[
            pltpu.VMEM((SUB,), jnp.int32),
            pltpu.VMEM((4, SUB), jnp.int32),
            pltpu.VMEM((SUB + 16,), jnp.int32),
            pltpu.VMEM((SUB, W2), f32),
            pltpu.VMEM((4 * SUB, W2), f32),
            pltpu.VMEM((SUB, D), f32),
            pltpu.SemaphoreType.DMA,
            pltpu.SemaphoreType.DMA,
        ],
    )
    def sc_k(a_hbm, m_hbm, nm_hbm, y_hbm, out_hbm,
             aidx, midx, nmbuf, bufA, mrows, outb, semA, semM):
        wid = lax.axis_index("s") * _NC + lax.axis_index("c")
        tile_base = wid * chunk

        def step_body(step, carry):
            base = tile_base + step * SUB
            pltpu.sync_copy(a_hbm.at[pl.ds(base, SUB)], aidx)
            pltpu.sync_copy(nm_hbm.at[pl.ds(base, SUB)], nmbuf.at[pl.ds(0, SUB)])
            for j in range(4):
                pltpu.sync_copy(m_hbm.at[j, pl.ds(base, SUB)], midx.at[j])
            _redirect_zeros(aidx, SUB, special)
            for j in range(4):
                for g in range(SUB // 16):
                    sl = pl.ds(g * 16, 16)
                    v = midx[j, sl]
                    midx[j, sl] = jnp.where(v == 0, special, v)

            cA = pltpu.async_copy(y_hbm.at[aidx], bufA, semA)
            cM = [pltpu.async_copy(y_hbm.at[midx.at[j]],
                                   mrows.at[pl.ds(j * SUB, SUB)], semM)
                  for j in range(4)]
            cA.wait()
            for c in cM:
                c.wait()

            third = jnp.float32(1.0 / 3.0)

            def row_body(t, carry2):
                nm = nmbuf[pl.ds(t, 16)][0]
                q = jnp.where(
                    nm <= 1, jnp.float32(1.0),
                    jnp.where(nm == 2, jnp.float32(0.5),
                              jnp.where(nm == 3, third, jnp.float32(0.25))))
                for g in range(D // 16):
                    sl = pl.ds(g * 16, 16)
                    sl_hi = pl.ds(D + g * 16, 16)
                    msum = (mrows[t, sl_hi] + mrows[SUB + t, sl_hi]
                            + mrows[2 * SUB + t, sl_hi]
                            + mrows[3 * SUB + t, sl_hi])
                    outb[t, sl] = bufA[t, sl] + msum * q
                return carry2
            lax.fori_loop(0, SUB, row_body, 0)
            pltpu.sync_copy(outb, out_hbm.at[pl.ds(base, SUB)])
            return carry

        lax.fori_loop(0, nstep, step_body, 0)

    return sc_k(a_tok, m_flat, num_moves, Y)


def _sc_gather_x(s_tok, i_tok, partialY, X, special):
    B = s_tok.shape[0]
    W2 = X.shape[1]
    D = W2 // 2
    f32 = jnp.float32
    SUB = 128
    chunk = B // _NW
    nstep = chunk // SUB

    @functools.partial(
        pl.kernel,
        out_type=jax.ShapeDtypeStruct((B, D), f32),
        mesh=_mesh(),
        compiler_params=pltpu.CompilerParams(use_tc_tiling_on_sc=True),
        scratch_types=[
            pltpu.VMEM((2, SUB), jnp.int32),
            pltpu.VMEM((2, SUB), jnp.int32),
            pltpu.VMEM((2 * SUB, W2), f32),
            pltpu.VMEM((2 * SUB, W2), f32),
            pltpu.VMEM((SUB, D), f32),
            pltpu.VMEM((SUB, D), f32),
            pltpu.SemaphoreType.DMA,
            pltpu.SemaphoreType.DMA,
            pltpu.SemaphoreType.DMA,
            pltpu.SemaphoreType.DMA,
        ],
    )
    def sc_k(s_hbm, i_hbm, p_hbm, x_hbm, out_hbm,
             sidx, iidx, bufS, bufI, pbuf, outb, semS0, semI0, semS1, semI1):
        wid = lax.axis_index("s") * _NC + lax.axis_index("c")
        tile_base = wid * chunk

        sems = [(semS0, semI0), (semS1, semI1)]

        def fire(step):
            par = step % 2
            base = tile_base + step * SUB
            semS, semI = sems[par]
            pltpu.sync_copy(s_hbm.at[pl.ds(base, SUB)], sidx.at[par])
            pltpu.sync_copy(i_hbm.at[pl.ds(base, SUB)], iidx.at[par])
            for g in range(SUB // 16):
                sl = pl.ds(g * 16, 16)
                v = sidx[par, sl]
                sidx[par, sl] = jnp.where(v == 0, special, v)
                w = iidx[par, sl]
                iidx[par, sl] = jnp.where(w == 0, special, w)
            half = pl.ds(par * SUB, SUB)
            cS = pltpu.async_copy(x_hbm.at[sidx.at[par]], bufS.at[half], semS)
            cI = pltpu.async_copy(x_hbm.at[iidx.at[par]], bufI.at[half], semI)
            return cS, cI

        def drain(step, cS, cI):
            par = step % 2
            base = tile_base + step * SUB
            pltpu.sync_copy(p_hbm.at[pl.ds(base, SUB)], pbuf)
            cS.wait()
            cI.wait()
            off = par * SUB

            def row_body(r, carry2):
                for g in range(D // 16):
                    sl = pl.ds(g * 16, 16)
                    sl_hi = pl.ds(D + g * 16, 16)
                    outb[r, sl] = (bufS[off + r, sl] + bufI[off + r, sl_hi]
                                   + pbuf[r, sl])
                return carry2
            lax.fori_loop(0, SUB, row_body, 0)
            pltpu.sync_copy(outb, out_hbm.at[pl.ds(base, SUB)])

        pending = fire(0)
        for step in range(nstep):
            nxt = fire(step + 1) if step + 1 < nstep else None
            drain(step, *pending)
            pending = nxt

    return sc_k(s_tok, i_tok, partialY, X)


def kernel(species_tokens, ability_tokens, item_tokens, move_tokens, num_moves,
           species_table, items_table, abilities_table, moves_table,
           pre_species, pre_items, pre_abilities, pre_moves,
           species_W, items_W, abilities_W, moves_W):
    m_t = move_tokens.T
    l0_x = jnp.concatenate([species_table[0:1], items_table[0:1]], axis=1)
    l0_y = jnp.concatenate([abilities_table[0:1], moves_table[0:1]], axis=1)
    Y = _tc_fold_pair(abilities_table.T, moves_table.T,
                      pre_abilities, pre_moves, abilities_W, moves_W, l0_y)
    X = _tc_fold_pair(species_table.T, items_table.T,
                      pre_species, pre_items, species_W, items_W, l0_x)
    V = species_table.shape[0]
    partialY = _sc_gather_y(ability_tokens, m_t, num_moves, Y, V)
    return _sc_gather_x(species_tokens, item_tokens, partialY, X, V)


# speedup vs baseline: 1.2902x; 1.0257x over previous
import functools

import jax
import jax.numpy as jnp
from jax import lax
from jax.experimental import pallas as pl
from jax.experimental.pallas import tpu as pltpu
from jax.experimental.pallas import tpu_sc as plsc

_NC, _NS = 2, 16
_NW = _NC * _NS
_VB = 8192


def _tc_fold_pair(lt1, lt2, pre1, pre2, W1, W2, l0pair):
    D, V = lt1.shape
    P = pre1.shape[1]
    grid_n = (V + _VB - 1) // _VB + (1 if V % _VB == 0 else 0)
    special = V
    VPAD = grid_n * _VB
    f32 = jnp.float32

    def body(lt1_r, lt2_r, p1_r, p2_r, w1_r, w2_r, l0_r, out_r):
        def fused(lt_r, p_r, w_r):
            lt = jnp.transpose(lt_r[...], (1, 0))
            return lt + jnp.dot(p_r[...], w_r[...], preferred_element_type=f32)
        full = jnp.concatenate(
            [fused(lt1_r, p1_r, w1_r), fused(lt2_r, p2_r, w2_r)], axis=1)
        rows = (pl.program_id(0) * _VB
                + jax.lax.broadcasted_iota(jnp.int32, (_VB, 1), 0))
        out_r[...] = jnp.where(rows == special, l0_r[...], full)

    blk_lt = pl.BlockSpec((D, _VB), lambda i: (0, i))
    blk_p = pl.BlockSpec((_VB, P), lambda i: (i, 0))
    blk_w = pl.BlockSpec((P, D), lambda i: (0, 0))
    blk_l0 = pl.BlockSpec((1, 2 * D), lambda i: (0, 0))
    blk_o = pl.BlockSpec((_VB, 2 * D), lambda i: (i, 0))

    return pl.pallas_call(
        body,
        grid=(grid_n,),
        in_specs=[blk_lt, blk_lt, blk_p, blk_p, blk_w, blk_w, blk_l0],
        out_specs=blk_o,
        out_shape=jax.ShapeDtypeStruct((VPAD, 2 * D), f32),
    )(lt1, lt2, pre1, pre2, W1, W2, l0pair)


def _mesh():
    return plsc.VectorSubcoreMesh(core_axis_name="c", subcore_axis_name="s",
                                  num_cores=_NC, num_subcores=_NS)


def _redirect_zeros(idx_ref, n, special):
    for g in range(n // 16):
        sl = pl.ds(g * 16, 16)
        v = idx_ref[sl]
        idx_ref[sl] = jnp.where(v == 0, special, v)


def _sc_gather_y(a_tok, m_flat, num_moves, Y, special):
    B = a_tok.shape[0]
    W2 = Y.shape[1]
    D = W2 // 2
    f32 = jnp.float32
    SUB = 128
    chunk = B // _NW
    nstep = chunk // SUB

    @functools.partial(
        pl.kernel,
        out_type=jax.ShapeDtypeStruct((B, D), f32),
        mesh=_mesh(),
        compiler_params=pltpu.CompilerParams(use_tc_tiling_on_sc=True),
        scratch_types=[
            pltpu.VMEM((SUB,), jnp.int32),
            pltpu.VMEM((4, SUB), jnp.int32),
            pltpu.VMEM((SUB + 16,), jnp.int32),
            pltpu.VMEM((SUB, W2), f32),
            pltpu.VMEM((4 * SUB, W2), f32),
            pltpu.VMEM((SUB, D), f32),
            pltpu.SemaphoreType.DMA,
            pltpu.SemaphoreType.DMA,
        ],
    )
    def sc_k(a_hbm, m_hbm, nm_hbm, y_hbm, out_hbm,
             aidx, midx, nmbuf, bufA, mrows, outb, semA, semM):
        wid = lax.axis_index("s") * _NC + lax.axis_index("c")
        tile_base = wid * chunk

        def step_body(step, carry):
            base = tile_base + step * SUB
            pltpu.sync_copy(a_hbm.at[pl.ds(base, SUB)], aidx)
            pltpu.sync_copy(nm_hbm.at[pl.ds(base, SUB)], nmbuf.at[pl.ds(0, SUB)])
            for j in range(4):
                pltpu.sync_copy(m_hbm.at[j, pl.ds(base, SUB)], midx.at[j])
            _redirect_zeros(aidx, SUB, special)
            for j in range(4):
                for g in range(SUB // 16):
                    sl = pl.ds(g * 16, 16)
                    v = midx[j, sl]
                    midx[j, sl] = jnp.where(v == 0, special, v)

            cA = pltpu.async_copy(y_hbm.at[aidx], bufA, semA)
            cM = [pltpu.async_copy(y_hbm.at[midx.at[j]],
                                   mrows.at[pl.ds(j * SUB, SUB)], semM)
                  for j in range(4)]
            cA.wait()
            for c in cM:
                c.wait()

            third = jnp.float32(1.0 / 3.0)

            def row_body(t, carry2):
                nm = nmbuf[pl.ds(t, 16)][0]
                q = jnp.where(
                    nm <= 1, jnp.float32(1.0),
                    jnp.where(nm == 2, jnp.float32(0.5),
                              jnp.where(nm == 3, third, jnp.float32(0.25))))
                for g in range(D // 16):
                    sl = pl.ds(g * 16, 16)
                    sl_hi = pl.ds(D + g * 16, 16)
                    msum = (mrows[t, sl_hi] + mrows[SUB + t, sl_hi]
                            + mrows[2 * SUB + t, sl_hi]
                            + mrows[3 * SUB + t, sl_hi])
                    outb[t, sl] = bufA[t, sl] + msum * q
                return carry2
            lax.fori_loop(0, SUB, row_body, 0)
            pltpu.sync_copy(outb, out_hbm.at[pl.ds(base, SUB)])
            return carry

        lax.fori_loop(0, nstep, step_body, 0)

    return sc_k(a_tok, m_flat, num_moves, Y)


def _sc_gather_x(s_tok, i_tok, partialY, X, special):
    B = s_tok.shape[0]
    W2 = X.shape[1]
    D = W2 // 2
    f32 = jnp.float32
    SUB = 64
    chunk = B // _NW
    nstep = chunk // SUB

    @functools.partial(
        pl.kernel,
        out_type=jax.ShapeDtypeStruct((B, D), f32),
        mesh=_mesh(),
        compiler_params=pltpu.CompilerParams(use_tc_tiling_on_sc=True),
        scratch_types=[
            pltpu.VMEM((2, SUB), jnp.int32),
            pltpu.VMEM((2, SUB), jnp.int32),
            pltpu.VMEM((2 * SUB, W2), f32),
            pltpu.VMEM((2 * SUB, W2), f32),
            pltpu.VMEM((2 * SUB, D), f32),
            pltpu.VMEM((SUB, D), f32),
            pltpu.SemaphoreType.DMA,
            pltpu.SemaphoreType.DMA,
            pltpu.SemaphoreType.DMA,
            pltpu.SemaphoreType.DMA,
            pltpu.SemaphoreType.DMA,
            pltpu.SemaphoreType.DMA,
        ],
    )
    def sc_k(s_hbm, i_hbm, p_hbm, x_hbm, out_hbm,
             sidx, iidx, bufS, bufI, pbuf, outb,
             semS0, semI0, semP0, semS1, semI1, semP1):
        wid = lax.axis_index("s") * _NC + lax.axis_index("c")
        tile_base = wid * chunk

        sems = [(semS0, semI0, semP0), (semS1, semI1, semP1)]

        def fire(step):
            par = step % 2
            base = tile_base + step * SUB
            semS, semI, semP = sems[par]
            half = pl.ds(par * SUB, SUB)
            cP = pltpu.async_copy(p_hbm.at[pl.ds(base, SUB)],
                                  pbuf.at[half], semP)
            pltpu.sync_copy(s_hbm.at[pl.ds(base, SUB)], sidx.at[par])
            pltpu.sync_copy(i_hbm.at[pl.ds(base, SUB)], iidx.at[par])
            for g in range(SUB // 16):
                sl = pl.ds(g * 16, 16)
                v = sidx[par, sl]
                sidx[par, sl] = jnp.where(v == 0, special, v)
                w = iidx[par, sl]
                iidx[par, sl] = jnp.where(w == 0, special, w)
            cS = pltpu.async_copy(x_hbm.at[sidx.at[par]], bufS.at[half], semS)
            cI = pltpu.async_copy(x_hbm.at[iidx.at[par]], bufI.at[half], semI)
            return cS, cI, cP

        def drain(step, cS, cI, cP):
            par = step % 2
            base = tile_base + step * SUB
            cS.wait()
            cI.wait()
            cP.wait()
            off = par * SUB

            def row_body(r, carry2):
                for g in range(D // 16):
                    sl = pl.ds(g * 16, 16)
                    sl_hi = pl.ds(D + g * 16, 16)
                    outb[r, sl] = (bufS[off + r, sl] + bufI[off + r, sl_hi]
                                   + pbuf[off + r, sl])
                return carry2
            lax.fori_loop(0, SUB, row_body, 0)
            pltpu.sync_copy(outb, out_hbm.at[pl.ds(base, SUB)])

        pending = fire(0)
        for step in range(nstep):
            nxt = fire(step + 1) if step + 1 < nstep else None
            drain(step, *pending)
            pending = nxt

    return sc_k(s_tok, i_tok, partialY, X)


def kernel(species_tokens, ability_tokens, item_tokens, move_tokens, num_moves,
           species_table, items_table, abilities_table, moves_table,
           pre_species, pre_items, pre_abilities, pre_moves,
           species_W, items_W, abilities_W, moves_W):
    m_t = move_tokens.T
    l0_x = jnp.concatenate([species_table[0:1], items_table[0:1]], axis=1)
    l0_y = jnp.concatenate([abilities_table[0:1], moves_table[0:1]], axis=1)
    Y = _tc_fold_pair(abilities_table.T, moves_table.T,
                      pre_abilities, pre_moves, abilities_W, moves_W, l0_y)
    X = _tc_fold_pair(species_table.T, items_table.T,
                      pre_species, pre_items, species_W, items_W, l0_x)
    V = species_table.shape[0]
    partialY = _sc_gather_y(ability_tokens, m_t, num_moves, Y, V)
    return _sc_gather_x(species_tokens, item_tokens, partialY, X, V)


# speedup vs baseline: 1.2945x; 1.0034x over previous
import functools

import jax
import jax.numpy as jnp
from jax import lax
from jax.experimental import pallas as pl
from jax.experimental.pallas import tpu as pltpu
from jax.experimental.pallas import tpu_sc as plsc

_NC, _NS = 2, 16
_NW = _NC * _NS
_VB = 8192


def _tc_fold_pair(lt1, lt2, pre1, pre2, W1, W2, l0pair):
    D, V = lt1.shape
    P = pre1.shape[1]
    grid_n = (V + _VB - 1) // _VB + (1 if V % _VB == 0 else 0)
    special = V
    VPAD = grid_n * _VB
    f32 = jnp.float32

    def body(lt1_r, lt2_r, p1_r, p2_r, w1_r, w2_r, l0_r, out_r):
        def fused(lt_r, p_r, w_r):
            lt = jnp.transpose(lt_r[...], (1, 0))
            return lt + jnp.dot(p_r[...], w_r[...], preferred_element_type=f32)
        full = jnp.concatenate(
            [fused(lt1_r, p1_r, w1_r), fused(lt2_r, p2_r, w2_r)], axis=1)
        rows = (pl.program_id(0) * _VB
                + jax.lax.broadcasted_iota(jnp.int32, (_VB, 1), 0))
        out_r[...] = jnp.where(rows == special, l0_r[...], full)

    blk_lt = pl.BlockSpec((D, _VB), lambda i: (0, i))
    blk_p = pl.BlockSpec((_VB, P), lambda i: (i, 0))
    blk_w = pl.BlockSpec((P, D), lambda i: (0, 0))
    blk_l0 = pl.BlockSpec((1, 2 * D), lambda i: (0, 0))
    blk_o = pl.BlockSpec((_VB, 2 * D), lambda i: (i, 0))

    return pl.pallas_call(
        body,
        grid=(grid_n,),
        in_specs=[blk_lt, blk_lt, blk_p, blk_p, blk_w, blk_w, blk_l0],
        out_specs=blk_o,
        out_shape=jax.ShapeDtypeStruct((VPAD, 2 * D), f32),
    )(lt1, lt2, pre1, pre2, W1, W2, l0pair)


def _mesh():
    return plsc.VectorSubcoreMesh(core_axis_name="c", subcore_axis_name="s",
                                  num_cores=_NC, num_subcores=_NS)


def _redirect_zeros(idx_ref, n, special):
    for g in range(n // 16):
        sl = pl.ds(g * 16, 16)
        v = idx_ref[sl]
        idx_ref[sl] = jnp.where(v == 0, special, v)


def _sc_gather_y(a_tok, m_flat, num_moves, Y, special):
    B = a_tok.shape[0]
    W2 = Y.shape[1]
    D = W2 // 2
    f32 = jnp.float32
    SUB = 64
    chunk = B // _NW
    nstep = chunk // SUB

    @functools.partial(
        pl.kernel,
        out_type=jax.ShapeDtypeStruct((B, D), f32),
        mesh=_mesh(),
        compiler_params=pltpu.CompilerParams(use_tc_tiling_on_sc=True),
        scratch_types=[
            pltpu.VMEM((2, SUB), jnp.int32),
            pltpu.VMEM((8, SUB), jnp.int32),
            pltpu.VMEM((2, SUB + 16), jnp.int32),
            pltpu.VMEM((2 * SUB, W2), f32),
            pltpu.VMEM((8 * SUB, W2), f32),
            pltpu.VMEM((SUB, D), f32),
            pltpu.SemaphoreType.DMA,
            pltpu.SemaphoreType.DMA,
            pltpu.SemaphoreType.DMA,
            pltpu.SemaphoreType.DMA,
        ],
    )
    def sc_k(a_hbm, m_hbm, nm_hbm, y_hbm, out_hbm,
             aidx, midx, nmbuf, bufA, mrows, outb,
             semA0, semM0, semA1, semM1):
        wid = lax.axis_index("s") * _NC + lax.axis_index("c")
        tile_base = wid * chunk

        sems = [(semA0, semM0), (semA1, semM1)]

        def fire(step):
            par = step % 2
            base = tile_base + step * SUB
            semA, semM = sems[par]
            pltpu.sync_copy(a_hbm.at[pl.ds(base, SUB)], aidx.at[par])
            pltpu.sync_copy(nm_hbm.at[pl.ds(base, SUB)],
                            nmbuf.at[par, pl.ds(0, SUB)])
            for j in range(4):
                pltpu.sync_copy(m_hbm.at[j, pl.ds(base, SUB)],
                                midx.at[par * 4 + j])
            for g in range(SUB // 16):
                sl = pl.ds(g * 16, 16)
                v = aidx[par, sl]
                aidx[par, sl] = jnp.where(v == 0, special, v)
                for j in range(4):
                    w = midx[par * 4 + j, sl]
                    midx[par * 4 + j, sl] = jnp.where(w == 0, special, w)

            cA = pltpu.async_copy(y_hbm.at[aidx.at[par]],
                                  bufA.at[pl.ds(par * SUB, SUB)], semA)
            cM = [pltpu.async_copy(
                      y_hbm.at[midx.at[par * 4 + j]],
                      mrows.at[pl.ds((par * 4 + j) * SUB, SUB)], semM)
                  for j in range(4)]
            return cA, cM

        third = jnp.float32(1.0 / 3.0)

        def drain(step, cA, cM):
            par = step % 2
            base = tile_base + step * SUB
            cA.wait()
            for c in cM:
                c.wait()
            off = par * SUB
            moff = par * 4 * SUB

            def row_body(t, carry2):
                nm = nmbuf[par, pl.ds(t, 16)][0]
                q = jnp.where(
                    nm <= 1, jnp.float32(1.0),
                    jnp.where(nm == 2, jnp.float32(0.5),
                              jnp.where(nm == 3, third, jnp.float32(0.25))))
                for g in range(D // 16):
                    sl = pl.ds(g * 16, 16)
                    sl_hi = pl.ds(D + g * 16, 16)
                    msum = (mrows[moff + t, sl_hi]
                            + mrows[moff + SUB + t, sl_hi]
                            + mrows[moff + 2 * SUB + t, sl_hi]
                            + mrows[moff + 3 * SUB + t, sl_hi])
                    outb[t, sl] = bufA[off + t, sl] + msum * q
                return carry2
            lax.fori_loop(0, SUB, row_body, 0)
            pltpu.sync_copy(outb, out_hbm.at[pl.ds(base, SUB)])

        pending = fire(0)
        for step in range(nstep):
            nxt = fire(step + 1) if step + 1 < nstep else None
            drain(step, *pending)
            pending = nxt

    return sc_k(a_tok, m_flat, num_moves, Y)


def _sc_gather_x(s_tok, i_tok, partialY, X, special):
    B = s_tok.shape[0]
    W2 = X.shape[1]
    D = W2 // 2
    f32 = jnp.float32
    SUB = 64
    chunk = B // _NW
    nstep = chunk // SUB

    @functools.partial(
        pl.kernel,
        out_type=jax.ShapeDtypeStruct((B, D), f32),
        mesh=_mesh(),
        compiler_params=pltpu.CompilerParams(use_tc_tiling_on_sc=True),
        scratch_types=[
            pltpu.VMEM((2, SUB), jnp.int32),
            pltpu.VMEM((2, SUB), jnp.int32),
            pltpu.VMEM((2 * SUB, W2), f32),
            pltpu.VMEM((2 * SUB, W2), f32),
            pltpu.VMEM((2 * SUB, D), f32),
            pltpu.VMEM((SUB, D), f32),
            pltpu.SemaphoreType.DMA,
            pltpu.SemaphoreType.DMA,
            pltpu.SemaphoreType.DMA,
            pltpu.SemaphoreType.DMA,
            pltpu.SemaphoreType.DMA,
            pltpu.SemaphoreType.DMA,
        ],
    )
    def sc_k(s_hbm, i_hbm, p_hbm, x_hbm, out_hbm,
             sidx, iidx, bufS, bufI, pbuf, outb,
             semS0, semI0, semP0, semS1, semI1, semP1):
        wid = lax.axis_index("s") * _NC + lax.axis_index("c")
        tile_base = wid * chunk

        sems = [(semS0, semI0, semP0), (semS1, semI1, semP1)]

        def fire(step):
            par = step % 2
            base = tile_base + step * SUB
            semS, semI, semP = sems[par]
            half = pl.ds(par * SUB, SUB)
            cP = pltpu.async_copy(p_hbm.at[pl.ds(base, SUB)],
                                  pbuf.at[half], semP)
            pltpu.sync_copy(s_hbm.at[pl.ds(base, SUB)], sidx.at[par])
            pltpu.sync_copy(i_hbm.at[pl.ds(base, SUB)], iidx.at[par])
            for g in range(SUB // 16):
                sl = pl.ds(g * 16, 16)
                v = sidx[par, sl]
                sidx[par, sl] = jnp.where(v == 0, special, v)
                w = iidx[par, sl]
                iidx[par, sl] = jnp.where(w == 0, special, w)
            cS = pltpu.async_copy(x_hbm.at[sidx.at[par]], bufS.at[half], semS)
            cI = pltpu.async_copy(x_hbm.at[iidx.at[par]], bufI.at[half], semI)
            return cS, cI, cP

        def drain(step, cS, cI, cP):
            par = step % 2
            base = tile_base + step * SUB
            cS.wait()
            cI.wait()
            cP.wait()
            off = par * SUB

            def row_body(r, carry2):
                for g in range(D // 16):
                    sl = pl.ds(g * 16, 16)
                    sl_hi = pl.ds(D + g * 16, 16)
                    outb[r, sl] = (bufS[off + r, sl] + bufI[off + r, sl_hi]
                                   + pbuf[off + r, sl])
                return carry2
            lax.fori_loop(0, SUB, row_body, 0)
            pltpu.sync_copy(outb, out_hbm.at[pl.ds(base, SUB)])

        pending = fire(0)
        for step in range(nstep):
            nxt = fire(step + 1) if step + 1 < nstep else None
            drain(step, *pending)
            pending = nxt

    return sc_k(s_tok, i_tok, partialY, X)


def kernel(species_tokens, ability_tokens, item_tokens, move_tokens, num_moves,
           species_table, items_table, abilities_table, moves_table,
           pre_species, pre_items, pre_abilities, pre_moves,
           species_W, items_W, abilities_W, moves_W):
    m_t = move_tokens.T
    l0_x = jnp.concatenate([species_table[0:1], items_table[0:1]], axis=1)
    l0_y = jnp.concatenate([abilities_table[0:1], moves_table[0:1]], axis=1)
    Y = _tc_fold_pair(abilities_table.T, moves_table.T,
                      pre_abilities, pre_moves, abilities_W, moves_W, l0_y)
    X = _tc_fold_pair(species_table.T, items_table.T,
                      pre_species, pre_items, species_W, items_W, l0_x)
    V = species_table.shape[0]
    partialY = _sc_gather_y(ability_tokens, m_t, num_moves, Y, V)
    return _sc_gather_x(species_tokens, item_tokens, partialY, X, V)
